# baseline (speedup 1.0000x reference)
import functools

import jax
import jax.numpy as jnp
from jax import lax
from jax.experimental import pallas as pl
from jax.experimental.pallas import tpu as pltpu
from jax.experimental.pallas import tpu_sc as plsc

_N = 10000
_E = 320000
_G = 8
_NPAD = 10240
_R = 512
_NB = _NPAD // _R
_NSC = 2
_NT = 16
_C = 128
_W = 128
_NCHA = 80
_IBA = 16
_EA = _NSC * _NT * _NCHA * _C
_NCHB = 160
_IBB = 32
_EB = _NT * _NCHB * _C
_RT = _NPAD // _NT

_f32 = jnp.float32
_PREC = lax.Precision.DEFAULT


def _dot(a, b):
    return lax.dot_general(a, b, (((1,), (0,)), ((), ())),
                           precision=_PREC, preferred_element_type=_f32)


def _mesh():
    return plsc.VectorSubcoreMesh(core_axis_name="c", subcore_axis_name="s",
                                  num_cores=_NSC, num_subcores=_NT)


def _full16(v):
    return jnp.full((16,), v, jnp.int32)


def _zero_rows(rows_v):
    z = jnp.zeros((16,), _f32)

    def zrow(r, _):
        for k in range(_W // 16):
            rows_v[r, pl.ds(k * 16, 16)] = z
        return 0

    lax.fori_loop(0, _C, zrow, 0)


def _zero_acc(rows_v, acc_sh, s):
    def zacc(j, _):
        pltpu.sync_copy(rows_v, acc_sh.at[pl.ds(s * _RT + j * _C, _C)])
        return 0

    lax.fori_loop(0, _RT // _C, zacc, 0)


def _writeback(acc_sh, rows_v, out_hbm_c, s):
    def wb(j, _):
        pltpu.sync_copy(acc_sh.at[pl.ds(s * _RT + j * _C, _C)], rows_v)
        pltpu.sync_copy(rows_v, out_hbm_c.at[pl.ds(s * _RT + j * _C, _C)])
        return 0

    lax.fori_loop(0, _RT // _C, wb, 0)


def _splat(vec16, m):
    return lax.gather(
        vec16, jnp.full((16, 1), m, jnp.int32),
        lax.GatherDimensionNumbers(offset_dims=(), collapsed_slice_dims=(0,),
                                   start_index_map=(0,)),
        (1,), mode=lax.GatherScatterMode.PROMISE_IN_BOUNDS)


def _scale_rows(rows_v, w_v, j):
    def emul(r, _):
        q16 = pl.multiple_of((r // 16) * 16, 16)
        wq = w_v[j, pl.ds(q16, 16)]
        splat = _splat(wq, r - q16)
        for k in range(_W // 16):
            rows_v[r, pl.ds(k * 16, 16)] = (
                rows_v[r, pl.ds(k * 16, 16)] * splat)
        return 0

    lax.fori_loop(0, _C, emul, 0)


@functools.partial(
    pl.kernel,
    out_type=jax.ShapeDtypeStruct((_NSC, _NPAD, _W), _f32),
    mesh=_mesh(),
    scratch_types=[
        pltpu.VMEM((_NCHA, _C), jnp.int32),
        pltpu.VMEM((_C, _W), _f32),
        pltpu.VMEM_SHARED((_NPAD, _W), _f32),
    ],
)
def _deg_kernel(src_hbm, out_hbm, src_v, val_v, acc_sh):
    c = lax.axis_index("c")
    s = lax.axis_index("s")
    w = c * _NT + s
    pltpu.sync_copy(src_hbm.at[w], src_v)
    _zero_rows(val_v)
    _zero_acc(val_v, acc_sh, s)
    plsc.subcore_barrier()

    one = jnp.ones((16,), _f32)

    def orow(r, _):
        for k in range(_W // 16):
            val_v[r, pl.ds(k * 16, 16)] = one
        return 0

    lax.fori_loop(0, _C, orow, 0)

    def step(j, _):
        pltpu.sync_copy(val_v, acc_sh.at[src_v.at[j]], add=True)
        return 0

    lax.fori_loop(0, _NCHA, step, 0)
    plsc.subcore_barrier()
    _zero_rows(val_v)
    _writeback(acc_sh, val_v, out_hbm.at[c], s)


@functools.partial(
    pl.kernel,
    out_type=jax.ShapeDtypeStruct((_NSC * _NT, _NCHA, _C), _f32),
    mesh=_mesh(),
    scratch_types=[
        pltpu.VMEM((_NCHA, _C), jnp.int32),
        pltpu.VMEM((_NCHA, _C), jnp.int32),
        pltpu.VMEM((_C,), _f32),
        pltpu.VMEM((_C,), _f32),
        pltpu.VMEM((_NCHA, _C), _f32),
    ],
)
def _wns_kernel(dinv_hbm, src_hbm, dst_hbm, out_hbm,
                src_v, dst_v, ds_v, dd_v, w_v):
    c = lax.axis_index("c")
    s = lax.axis_index("s")
    w = c * _NT + s
    pltpu.sync_copy(src_hbm.at[w], src_v)
    pltpu.sync_copy(dst_hbm.at[w], dst_v)

    def chunk(j, _):
        pltpu.sync_copy(dinv_hbm.at[src_v.at[j]], ds_v)
        pltpu.sync_copy(dinv_hbm.at[dst_v.at[j]], dd_v)

        def sub(q, _):
            w_v[j, pl.ds(q * 16, 16)] = -(ds_v[pl.ds(q * 16, 16)]
                                          * dd_v[pl.ds(q * 16, 16)])
            return 0

        lax.fori_loop(0, _C // 16, sub, 0)
        return 0

    lax.fori_loop(0, _NCHA, chunk, 0)
    pltpu.sync_copy(w_v, out_hbm.at[w])


@functools.partial(
    pl.kernel,
    out_type=jax.ShapeDtypeStruct((_NSC, _NPAD, _W), _f32),
    mesh=_mesh(),
    scratch_types=[
        pltpu.VMEM((_IBA, _C), jnp.int32),
        pltpu.VMEM((_IBA, _C), jnp.int32),
        pltpu.VMEM((_IBA, _C), _f32),
        pltpu.VMEM((_C, _W), _f32),
        pltpu.VMEM_SHARED((_NPAD, _W), _f32),
    ],
)
def _prop_edge_split(g_hbm, src_hbm, dst_hbm, wns_hbm, out_hbm,
                     src_v, dst_v, w_v, rows_v, acc_sh):
    c = lax.axis_index("c")
    s = lax.axis_index("s")
    w = c * _NT + s
    _zero_rows(rows_v)
    _zero_acc(rows_v, acc_sh, s)
    plsc.subcore_barrier()

    def blk(bi, _):
        pltpu.sync_copy(src_hbm.at[w].at[pl.ds(bi * _IBA, _IBA)], src_v)
        pltpu.sync_copy(dst_hbm.at[w].at[pl.ds(bi * _IBA, _IBA)], dst_v)
        pltpu.sync_copy(wns_hbm.at[w].at[pl.ds(bi * _IBA, _IBA)], w_v)

        def step(j, _):
            pltpu.sync_copy(g_hbm.at[src_v.at[j]], rows_v)
            _scale_rows(rows_v, w_v, j)
            pltpu.sync_copy(rows_v, acc_sh.at[dst_v.at[j]], add=True)
            return 0

        lax.fori_loop(0, _IBA, step, 0)
        return 0

    lax.fori_loop(0, _NCHA // _IBA, blk, 0)
    plsc.subcore_barrier()
    _writeback(acc_sh, rows_v, out_hbm.at[c], s)


@functools.partial(
    pl.kernel,
    out_type=jax.ShapeDtypeStruct((_NSC, _NPAD, _W), _f32),
    mesh=_mesh(),
    scratch_types=[
        pltpu.VMEM((_IBB, _C), jnp.int32),
        pltpu.VMEM((_IBB, _C), jnp.int32),
        pltpu.VMEM((_IBB, _C), _f32),
        pltpu.VMEM((_C, _W), _f32),
        pltpu.VMEM_SHARED((_NPAD, _W), _f32),
    ],
)
def _prop_feat_split(g_hbm, src_hbm, dst_hbm, wns_hbm, out_hbm,
                     src_v, dst_v, w_v, rows_v, acc_sh):
    c = lax.axis_index("c")
    s = lax.axis_index("s")
    _zero_rows(rows_v)
    _zero_acc(rows_v, acc_sh, s)
    plsc.subcore_barrier()

    def blk(bi, _):
        pltpu.sync_copy(src_hbm.at[s].at[pl.ds(bi * _IBB, _IBB)], src_v)
        pltpu.sync_copy(dst_hbm.at[s].at[pl.ds(bi * _IBB, _IBB)], dst_v)
        pltpu.sync_copy(wns_hbm.at[s].at[pl.ds(bi * _IBB, _IBB)], w_v)

        def step(j, _):
            pltpu.sync_copy(g_hbm.at[c].at[src_v.at[j]], rows_v)
            _scale_rows(rows_v, w_v, j)
            pltpu.sync_copy(rows_v, acc_sh.at[dst_v.at[j]], add=True)
            return 0

        lax.fori_loop(0, _IBB, step, 0)
        return 0

    lax.fori_loop(0, _NCHB // _IBB, blk, 0)
    plsc.subcore_barrier()
    _writeback(acc_sh, rows_v, out_hbm.at[c], s)


def _pre_body(deg_ref, dinv_ref):
    i = pl.program_id(0)
    deg = deg_ref[0, :, 0:1] + deg_ref[1, :, 0:1]
    rows = i * _R + lax.broadcasted_iota(jnp.int32, (_R, 1), 0)
    valid = (rows < _N) & (deg > 0)
    dinv_ref[...] = jnp.where(valid, 1.0 / jnp.sqrt(jnp.maximum(deg, 1.0)),
                              0.0)


def _pre_call(degp):
    return pl.pallas_call(
        _pre_body,
        grid=(_NB,),
        in_specs=[pl.BlockSpec((_NSC, _R, _W), lambda i: (0, i, 0))],
        out_specs=pl.BlockSpec((_R, 1), lambda i: (i, 0)),
        out_shape=jax.ShapeDtypeStruct((_NPAD, 1), _f32),
    )(degp)


def _mid_a_body(h_ref, a_ref, w0_ref, w1_ref, part_ref, t1_ref):
    t1 = a_ref[0] + a_ref[1]
    part_ref[...] = _dot(h_ref[...], w0_ref[...]) + _dot(t1, w1_ref[...])
    t1_ref[...] = t1


def _mid_a_call(h, a1, w0, w1):
    din, dout = w0.shape
    return pl.pallas_call(
        _mid_a_body,
        grid=(_NB,),
        in_specs=[
            pl.BlockSpec((_R, din), lambda i: (i, 0)),
            pl.BlockSpec((_NSC, _R, _W), lambda i: (0, i, 0)),
            pl.BlockSpec((din, dout), lambda i: (0, 0)),
            pl.BlockSpec((din, dout), lambda i: (0, 0)),
        ],
        out_specs=[
            pl.BlockSpec((_R, dout), lambda i: (i, 0)),
            pl.BlockSpec((_R, din), lambda i: (i, 0)),
        ],
        out_shape=[
            jax.ShapeDtypeStruct((_NPAD, dout), _f32),
            jax.ShapeDtypeStruct((_NPAD, din), _f32),
        ],
    )(h, a1, w0, w1)


def _mid_b_body(h_ref, a_ref, w0_ref, w1_ref, part_ref):
    t1 = jnp.concatenate([a_ref[0], a_ref[1]], axis=1)
    part_ref[...] = _dot(h_ref[...], w0_ref[...]) + _dot(t1, w1_ref[...])


def _mid_b_call(h, a1, w0, w1):
    din, dout = w0.shape
    return pl.pallas_call(
        _mid_b_body,
        grid=(_NB,),
        in_specs=[
            pl.BlockSpec((_R, din), lambda i: (i, 0)),
            pl.BlockSpec((_NSC, _R, _W), lambda i: (0, i, 0)),
            pl.BlockSpec((din, dout), lambda i: (0, 0)),
            pl.BlockSpec((din, dout), lambda i: (0, 0)),
        ],
        out_specs=pl.BlockSpec((_R, dout), lambda i: (i, 0)),
        out_shape=jax.ShapeDtypeStruct((_NPAD, dout), _f32),
    )(h, a1, w0, w1)


def _fin_a_body(h_ref, a_ref, part_ref, w2_ref, b_ref, hn_ref, hs_ref=None):
    tx2 = 2.0 * (a_ref[0] + a_ref[1]) - h_ref[...]
    hn = jnp.maximum(part_ref[...] + _dot(tx2, w2_ref[...]) + b_ref[...],
                     0.0)
    hn_ref[...] = hn
    if hs_ref is not None:
        hs_ref[0] = hn[:, :_W]
        hs_ref[1] = hn[:, _W:]


def _fin_a_call(h, a2, part, w2, b, split_out):
    din, dout = w2.shape
    out_specs = [pl.BlockSpec((_R, dout), lambda i: (i, 0))]
    out_shape = [jax.ShapeDtypeStruct((_NPAD, dout), _f32)]
    if split_out:
        out_specs.append(pl.BlockSpec((_NSC, _R, _W), lambda i: (0, i, 0)))
        out_shape.append(jax.ShapeDtypeStruct((_NSC, _NPAD, _W), _f32))
        body = _fin_a_body
    else:
        def body(h_ref, a_ref, part_ref, w2_ref, b_ref, hn_ref):
            _fin_a_body(h_ref, a_ref, part_ref, w2_ref, b_ref, hn_ref)
    res = pl.pallas_call(
        body,
        grid=(_NB,),
        in_specs=[
            pl.BlockSpec((_R, din), lambda i: (i, 0)),
            pl.BlockSpec((_NSC, _R, _W), lambda i: (0, i, 0)),
            pl.BlockSpec((_R, dout), lambda i: (i, 0)),
            pl.BlockSpec((din, dout), lambda i: (0, 0)),
            pl.BlockSpec((1, dout), lambda i: (0, 0)),
        ],
        out_specs=out_specs,
        out_shape=out_shape,
    )(h, a2, part, w2, b)
    return res if split_out else (res[0], None)


def _fin_b_body(h_ref, a_ref, part_ref, w2_ref, b_ref, hn_ref):
    t2 = jnp.concatenate([a_ref[0], a_ref[1]], axis=1)
    tx2 = 2.0 * t2 - h_ref[...]
    hn_ref[...] = jnp.maximum(
        part_ref[...] + _dot(tx2, w2_ref[...]) + b_ref[...], 0.0)


def _fin_b_call(h, a2, part, w2, b):
    din, dout = w2.shape
    return pl.pallas_call(
        _fin_b_body,
        grid=(_NB,),
        in_specs=[
            pl.BlockSpec((_R, din), lambda i: (i, 0)),
            pl.BlockSpec((_NSC, _R, _W), lambda i: (0, i, 0)),
            pl.BlockSpec((_R, dout), lambda i: (i, 0)),
            pl.BlockSpec((din, dout), lambda i: (0, 0)),
            pl.BlockSpec((1, dout), lambda i: (0, 0)),
        ],
        out_specs=pl.BlockSpec((_R, dout), lambda i: (i, 0)),
        out_shape=jax.ShapeDtypeStruct((_NPAD, dout), _f32),
    )(h, a2, part, w2, b)


def _head_body(h_ref, ids_ref, wl1_ref, bl1_ref, bng_ref, bnb_ref,
               wl2_ref, bl2_ref, out_ref, sig_ref, ssum, scnt):
    i = pl.program_id(0)

    @pl.when(i == 0)
    def _():
        ssum[...] = jnp.zeros_like(ssum)
        scnt[...] = jnp.zeros_like(scnt)

    ids = ids_ref[...]
    h = h_ref[...]
    oh = (ids == lax.broadcasted_iota(jnp.int32, (1, _G), 1)).astype(_f32)
    psum = jnp.stack([jnp.sum(jnp.where(ids == g, h, 0.0), axis=0)
                      for g in range(_G)])
    ssum[...] += psum
    scnt[...] += jnp.broadcast_to(jnp.sum(oh, axis=0)[:, None], scnt.shape)

    @pl.when(i == _NB - 1)
    def _():
        cnt = scnt[:, 0:1]
        xs = ssum[...] / jnp.maximum(cnt, 1.0)
        u = jnp.maximum(_dot(xs, wl1_ref[...]) + bl1_ref[...], 0.0)
        m = jnp.mean(u, axis=0, keepdims=True)
        d = u - m
        v = jnp.mean(d * d, axis=0, keepdims=True)
        bn = d / jnp.sqrt(v + 1e-5) * bng_ref[...] + bnb_ref[...]
        o = _dot(bn, wl2_ref[...]) + bl2_ref[...]
        out_ref[...] = o
        sig_ref[...] = 1.0 / (1.0 + jnp.exp(-o))


def _head_call(h3, ids, wl1, bl1, bng, bnb, wl2p, bl2p):
    dpad = wl2p.shape[1]
    return pl.pallas_call(
        _head_body,
        grid=(_NB,),
        in_specs=[
            pl.BlockSpec((_R, 512), lambda i: (i, 0)),
            pl.BlockSpec((_R, 1), lambda i: (i, 0)),
            pl.BlockSpec((512, 1024), lambda i: (0, 0)),
            pl.BlockSpec((1, 1024), lambda i: (0, 0)),
            pl.BlockSpec((1, 1024), lambda i: (0, 0)),
            pl.BlockSpec((1, 1024), lambda i: (0, 0)),
            pl.BlockSpec((1024, dpad), lambda i: (0, 0)),
            pl.BlockSpec((1, dpad), lambda i: (0, 0)),
        ],
        out_specs=[
            pl.BlockSpec((_G, dpad), lambda i: (0, 0)),
            pl.BlockSpec((_G, dpad), lambda i: (0, 0)),
        ],
        out_shape=[
            jax.ShapeDtypeStruct((_G, dpad), _f32),
            jax.ShapeDtypeStruct((_G, dpad), _f32),
        ],
        scratch_shapes=[
            pltpu.VMEM((_G, 512), _f32),
            pltpu.VMEM((_G, 128), _f32),
        ],
    )(h3, ids, wl1, bl1, bng, bnb, wl2p, bl2p)


def kernel(x_s, x_t, edge_index_s, edge_index_t, xs_batch, xt_batch,
           Wx1, bx1, Wx2, bx2, Wx3, bx3, Wy1, by1, Wy2, by2, Wy3, by3,
           W_l1, b_l1, bn_g, bn_b, W_l2, b_l2):
    n = x_s.shape[0]
    x = jnp.pad(x_s, ((0, _NPAD - n), (0, 0)))
    ids = jnp.pad(xs_batch, (0, _NPAD - n),
                  constant_values=_G).reshape(_NPAD, 1)
    dst, src = lax.sort((edge_index_s[1], edge_index_s[0]),
                        num_keys=1, is_stable=True)
    pad_idx = _NPAD - 1
    srcA = jnp.pad(src, (0, _EA - _E),
                   constant_values=pad_idx).reshape(_NSC * _NT, _NCHA, _C)
    dstA = jnp.pad(dst, (0, _EA - _E),
                   constant_values=pad_idx).reshape(_NSC * _NT, _NCHA, _C)
    srcB = jnp.pad(src, (0, _EB - _E),
                   constant_values=pad_idx).reshape(_NT, _NCHB, _C)
    dstB = jnp.pad(dst, (0, _EB - _E),
                   constant_values=pad_idx).reshape(_NT, _NCHB, _C)

    degp = _deg_kernel(srcA)
    dinv = _pre_call(degp)
    wA = _wns_kernel(dinv.reshape(_NPAD), srcA, dstA)
    wB = jnp.pad(wA.reshape(-1)[:_E], (0, _EB - _E)).reshape(
        _NT, _NCHB, _C)

    a1 = _prop_edge_split(x, srcA, dstA, wA)
    part, t1 = _mid_a_call(x, a1, Wx1[0], Wx1[1])
    a2 = _prop_edge_split(t1, srcA, dstA, wA)
    h, _ = _fin_a_call(x, a2, part, Wx1[2], bx1.reshape(1, -1), False)

    a1 = _prop_edge_split(h, srcA, dstA, wA)
    part, t1 = _mid_a_call(h, a1, Wx2[0], Wx2[1])
    a2 = _prop_edge_split(t1, srcA, dstA, wA)
    h, hsplit = _fin_a_call(h, a2, part, Wx2[2], bx2.reshape(1, -1), True)

    a1 = _prop_feat_split(hsplit, srcB, dstB, wB)
    part = _mid_b_call(h, a1, Wx3[0], Wx3[1])
    a2 = _prop_feat_split(a1, srcB, dstB, wB)
    h = _fin_b_call(h, a2, part, Wx3[2], bx3.reshape(1, -1))

    dpad = 1408
    wl2p = jnp.pad(W_l2, ((0, 0), (0, dpad - W_l2.shape[1])))
    bl2p = jnp.pad(b_l2, (0, dpad - b_l2.shape[0])).reshape(1, dpad)
    out, sig = _head_call(h, ids, W_l1, b_l1.reshape(1, -1),
                          bn_g.reshape(1, -1), bn_b.reshape(1, -1),
                          wl2p, bl2p)
    return out[:, :1317], sig[:, :1317]
